# baseline (speedup 1.0000x reference)
import functools

import jax
import jax.numpy as jnp
from jax import lax
from jax.experimental import pallas as pl
from jax.experimental.pallas import tpu as pltpu
from jax.experimental.pallas import tpu_sc as plsc

B = 2048
D = 256
R = 256
NB = B // R
GAMMA = 0.8
EPS = 1e-14
RHO = 8.0
GRAD_CLIP = 5.0
ETA_INIT = 1e-05

try:
    _SC_INFO = plsc.get_sparse_core_info()
    _NC, _NS = _SC_INFO.num_cores, _SC_INFO.num_subcores
except ValueError:
    _NC, _NS = 2, 16
_NW = _NC * _NS
_BPW = B // _NW

_vec2 = jax.ShapeDtypeStruct((2, B), jnp.float32)


@functools.partial(
    pl.kernel,
    mesh=plsc.VectorSubcoreMesh(core_axis_name="c", subcore_axis_name="s",
                                num_cores=_NC, num_subcores=_NS),
    out_type=[_vec2] * 3,
    scratch_types=[
        pltpu.VMEM((_BPW,), jnp.int32),
        pltpu.VMEM((_BPW,), jnp.int32),
        pltpu.VMEM((_BPW,), jnp.float32),
        pltpu.VMEM((_BPW,), jnp.float32),
        pltpu.VMEM((_BPW,), jnp.float32),
        pltpu.VMEM((_BPW,), jnp.float32),
        pltpu.VMEM((_BPW,), jnp.float32),
        pltpu.VMEM((_BPW,), jnp.float32),
        pltpu.SemaphoreType.DMA,
    ],
)
def _gather6(img_ids, txt_ids, tau_i_t, s_i_t, b_i_t, tau_t_t, s_t_t, b_t_t,
             o_tau, o_s, o_b, idx_i, idx_t, b0, b1, b2, b3, b4, b5, sem):
    wid = lax.axis_index("s") * _NC + lax.axis_index("c")
    base = wid * _BPW
    pltpu.sync_copy(img_ids.at[pl.ds(base, _BPW)], idx_i)
    pltpu.sync_copy(txt_ids.at[pl.ds(base, _BPW)], idx_t)
    plan = ((0, idx_i, tau_i_t, o_tau, b0), (0, idx_i, s_i_t, o_s, b1),
            (0, idx_i, b_i_t, o_b, b2), (1, idx_t, tau_t_t, o_tau, b3),
            (1, idx_t, s_t_t, o_s, b4), (1, idx_t, b_t_t, o_b, b5))
    copies = [pltpu.async_copy(table.at[idx], buf, sem)
              for row, idx, table, out, buf in plan]
    for c, (row, idx, table, out, buf) in zip(copies, plan):
        c.wait()
        pltpu.sync_copy(buf, out.at[row, pl.ds(base, _BPW)])


def _tc_body(img_ref, txt_ref, tau_ref, s_ref, b_ref, e0_ref,
             loss_ref, taui_ref, taut_ref, twim_ref, twtm_ref,
             twmax_ref, twmin_ref, OT_s, acc):
    s = pl.program_id(0)
    i = pl.program_id(1)
    ii = i * R
    is_img = s == 0

    @pl.when(i == 0)
    def _build_ot():
        def _t_img():
            return txt_ref[...].T

        def _t_txt():
            return img_ref[...].T

        OT_s[...] = lax.cond(is_img, _t_img, _t_txt)

    def _img_side():
        return (img_ref[pl.ds(ii, R), :],
                tau_ref[0, pl.ds(ii, R)], s_ref[0, pl.ds(ii, R)],
                b_ref[0, pl.ds(ii, R)])

    def _txt_side():
        return (txt_ref[pl.ds(ii, R), :],
                tau_ref[1, pl.ds(ii, R)], s_ref[1, pl.ds(ii, R)],
                b_ref[1, pl.ds(ii, R)])

    feat, tau, s_old, b_old = lax.cond(is_img, _img_side, _txt_side)
    S2 = jnp.dot(feat, OT_s[...], preferred_element_type=jnp.float32)
    col = lax.broadcasted_iota(jnp.int32, (R, B), 1)
    row = lax.broadcasted_iota(jnp.int32, (R, B), 0)
    is_diag = col == ii + row
    d = jnp.sum(jnp.where(is_diag, S2, 0.0), axis=1)
    rtau = 1.0 / tau
    m = jnp.max(S2, axis=1)
    b_new = jnp.maximum(b_old, (m - d) * rtau)
    diffs = S2 - d[:, None]
    E = jnp.exp(diffs * rtau[:, None] - b_new[:, None])
    E = jnp.where(is_diag, 0.0, E)
    g = jnp.sum(E, axis=1)
    P1 = jnp.sum(E * diffs, axis=1)
    ema = (1.0 - GAMMA) * s_old * jnp.exp(b_old - b_new) + GAMMA * g
    e0 = e0_ref[0, 0]
    sI = e0 * g + (1.0 - e0) * ema
    sIc = jnp.maximum(sI, EPS)
    rs = 1.0 / sIc
    loss_rows = P1 * rs
    wid_rows = loss_rows * rtau
    tw = jnp.log(sIc / (B - 1)) + b_new + RHO - wid_rows
    tw = jnp.clip(tw, -GRAD_CLIP, GRAD_CLIP)

    blk_loss = jnp.sum(loss_rows)
    blk_twsum = jnp.sum(tw)
    blk_twmax = jnp.max(tw)
    blk_twmin = jnp.min(tw)
    blk_tau = jnp.sum(tau)

    @pl.when((s == 0) & (i == 0))
    def _init():
        acc[0] = blk_loss
        acc[1] = blk_twsum
        acc[2] = 0.0
        acc[3] = blk_twmax
        acc[4] = blk_twmin
        acc[5] = blk_tau
        acc[6] = 0.0

    @pl.when((s != 0) | (i != 0))
    def _accum():
        acc[0] += blk_loss
        acc[1 + s] += blk_twsum
        acc[5 + s] += blk_tau

        @pl.when(s == 0)
        def _mm():
            acc[3] = jnp.maximum(acc[3], blk_twmax)
            acc[4] = jnp.minimum(acc[4], blk_twmin)

    @pl.when((s == 1) & (i == NB - 1))
    def _final():
        invB = jnp.float32(1.0 / B)
        loss_ref[0, 0] = acc[0] * invB
        taui_ref[0, 0] = acc[5] * invB
        taut_ref[0, 0] = acc[6] * invB
        twim_ref[0, 0] = acc[1] * invB
        twtm_ref[0, 0] = acc[2] * invB
        twmax_ref[0, 0] = acc[3]
        twmin_ref[0, 0] = acc[4]


_scal = jax.ShapeDtypeStruct((1, 1), jnp.float32)


def kernel(image_features, text_features, image_ids, text_ids, epoch, max_epoch,
           s_I, s_T, b_I, b_T, u_I, u_T, tau_I, tau_T, mask_neg):
    tau2, s2, b2 = _gather6(image_ids, text_ids, tau_I, s_I, b_I, tau_T, s_T, b_T)

    e0 = (jnp.asarray(epoch) == 0).astype(jnp.float32).reshape(1, 1)

    smem = pltpu.MemorySpace.SMEM
    full = lambda shape: pl.BlockSpec(shape, lambda s, i: tuple(0 for _ in shape))
    outs = pl.pallas_call(
        _tc_body,
        grid=(2, NB),
        in_specs=[
            full((B, D)),
            full((B, D)),
            full((2, B)),
            full((2, B)),
            full((2, B)),
            pl.BlockSpec(memory_space=smem),
        ],
        out_specs=[pl.BlockSpec((1, 1), lambda s, i: (0, 0), memory_space=smem)] * 7,
        out_shape=[_scal] * 7,
        scratch_shapes=[pltpu.VMEM((D, B), jnp.float32),
                        pltpu.SMEM((8,), jnp.float32)],
    )(image_features, text_features, tau2, s2, b2, e0)
    loss, taui, taut, twim, twtm, twmax, twmin = outs

    return (loss[0, 0], taui[0, 0], taut[0, 0], jnp.float32(ETA_INIT),
            twim[0, 0], twtm[0, 0], twmax[0, 0], twmin[0, 0])


# speedup vs baseline: 1.0850x; 1.0850x over previous
import functools

import jax
import jax.numpy as jnp
from jax import lax
from jax.experimental import pallas as pl
from jax.experimental.pallas import tpu as pltpu
from jax.experimental.pallas import tpu_sc as plsc

B = 2048
D = 256
R = 512
NB = B // R
GAMMA = 0.8
EPS = 1e-14
RHO = 8.0
GRAD_CLIP = 5.0
ETA_INIT = 1e-05

try:
    _SC_INFO = plsc.get_sparse_core_info()
    _NC, _NS = _SC_INFO.num_cores, _SC_INFO.num_subcores
except ValueError:
    _NC, _NS = 2, 16
_NW = _NC * _NS
_BPW = B // _NW

_vec2 = jax.ShapeDtypeStruct((2, B), jnp.float32)


@functools.partial(
    pl.kernel,
    mesh=plsc.VectorSubcoreMesh(core_axis_name="c", subcore_axis_name="s",
                                num_cores=_NC, num_subcores=_NS),
    out_type=[_vec2] * 3,
    scratch_types=[
        pltpu.VMEM((_BPW,), jnp.int32),
        pltpu.VMEM((_BPW,), jnp.int32),
        pltpu.VMEM((_BPW,), jnp.float32),
        pltpu.VMEM((_BPW,), jnp.float32),
        pltpu.VMEM((_BPW,), jnp.float32),
        pltpu.VMEM((_BPW,), jnp.float32),
        pltpu.VMEM((_BPW,), jnp.float32),
        pltpu.VMEM((_BPW,), jnp.float32),
        pltpu.SemaphoreType.DMA,
    ],
)
def _gather6(img_ids, txt_ids, tau_i_t, s_i_t, b_i_t, tau_t_t, s_t_t, b_t_t,
             o_tau, o_s, o_b, idx_i, idx_t, b0, b1, b2, b3, b4, b5, sem):
    wid = lax.axis_index("s") * _NC + lax.axis_index("c")
    base = wid * _BPW
    pltpu.sync_copy(img_ids.at[pl.ds(base, _BPW)], idx_i)
    pltpu.sync_copy(txt_ids.at[pl.ds(base, _BPW)], idx_t)
    plan = ((0, idx_i, tau_i_t, o_tau, b0), (0, idx_i, s_i_t, o_s, b1),
            (0, idx_i, b_i_t, o_b, b2), (1, idx_t, tau_t_t, o_tau, b3),
            (1, idx_t, s_t_t, o_s, b4), (1, idx_t, b_t_t, o_b, b5))
    copies = [pltpu.async_copy(table.at[idx], buf, sem)
              for row, idx, table, out, buf in plan]
    for c, (row, idx, table, out, buf) in zip(copies, plan):
        c.wait()
        pltpu.sync_copy(buf, out.at[row, pl.ds(base, _BPW)])


def _tc_body(img_ref, txt_ref, tau_ref, s_ref, b_ref, e0_ref,
             loss_ref, taui_ref, taut_ref, twim_ref, twtm_ref,
             twmax_ref, twmin_ref, OT_s, acc):
    s = pl.program_id(0)
    i = pl.program_id(1)
    ii = i * R
    is_img = s == 0

    @pl.when(i == 0)
    def _build_ot():
        def _t_img():
            return txt_ref[...].T

        def _t_txt():
            return img_ref[...].T

        OT_s[...] = lax.cond(is_img, _t_img, _t_txt)

    def _img_side():
        return (img_ref[pl.ds(ii, R), :],
                tau_ref[0, pl.ds(ii, R)], s_ref[0, pl.ds(ii, R)],
                b_ref[0, pl.ds(ii, R)])

    def _txt_side():
        return (txt_ref[pl.ds(ii, R), :],
                tau_ref[1, pl.ds(ii, R)], s_ref[1, pl.ds(ii, R)],
                b_ref[1, pl.ds(ii, R)])

    feat, tau, s_old, b_old = lax.cond(is_img, _img_side, _txt_side)
    S2 = jnp.dot(feat, OT_s[...], preferred_element_type=jnp.float32)
    col = lax.broadcasted_iota(jnp.int32, (R, B), 1)
    row = lax.broadcasted_iota(jnp.int32, (R, B), 0)
    is_diag = col == ii + row
    d = jnp.sum(jnp.where(is_diag, S2, 0.0), axis=1)
    rtau = 1.0 / tau
    m = jnp.max(S2, axis=1)
    b_new = jnp.maximum(b_old, (m - d) * rtau)
    diffs = S2 - d[:, None]
    E = jnp.exp(diffs * rtau[:, None] - b_new[:, None])
    E = jnp.where(is_diag, 0.0, E)
    g = jnp.sum(E, axis=1)
    P1 = jnp.sum(E * diffs, axis=1)
    ema = (1.0 - GAMMA) * s_old * jnp.exp(b_old - b_new) + GAMMA * g
    e0 = e0_ref[0, 0]
    sI = e0 * g + (1.0 - e0) * ema
    sIc = jnp.maximum(sI, EPS)
    rs = 1.0 / sIc
    loss_rows = P1 * rs
    wid_rows = loss_rows * rtau
    tw = jnp.log(sIc / (B - 1)) + b_new + RHO - wid_rows
    tw = jnp.clip(tw, -GRAD_CLIP, GRAD_CLIP)

    blk_loss = jnp.sum(loss_rows)
    blk_twsum = jnp.sum(tw)
    blk_twmax = jnp.max(tw)
    blk_twmin = jnp.min(tw)
    blk_tau = jnp.sum(tau)

    @pl.when((s == 0) & (i == 0))
    def _init():
        acc[0] = blk_loss
        acc[1] = blk_twsum
        acc[2] = 0.0
        acc[3] = blk_twmax
        acc[4] = blk_twmin
        acc[5] = blk_tau
        acc[6] = 0.0

    @pl.when((s != 0) | (i != 0))
    def _accum():
        acc[0] += blk_loss
        acc[1 + s] += blk_twsum
        acc[5 + s] += blk_tau

        @pl.when(s == 0)
        def _mm():
            acc[3] = jnp.maximum(acc[3], blk_twmax)
            acc[4] = jnp.minimum(acc[4], blk_twmin)

    @pl.when((s == 1) & (i == NB - 1))
    def _final():
        invB = jnp.float32(1.0 / B)
        loss_ref[0, 0] = acc[0] * invB
        taui_ref[0, 0] = acc[5] * invB
        taut_ref[0, 0] = acc[6] * invB
        twim_ref[0, 0] = acc[1] * invB
        twtm_ref[0, 0] = acc[2] * invB
        twmax_ref[0, 0] = acc[3]
        twmin_ref[0, 0] = acc[4]


_scal = jax.ShapeDtypeStruct((1, 1), jnp.float32)


def kernel(image_features, text_features, image_ids, text_ids, epoch, max_epoch,
           s_I, s_T, b_I, b_T, u_I, u_T, tau_I, tau_T, mask_neg):
    tau2, s2, b2 = _gather6(image_ids, text_ids, tau_I, s_I, b_I, tau_T, s_T, b_T)

    e0 = (jnp.asarray(epoch) == 0).astype(jnp.float32).reshape(1, 1)

    smem = pltpu.MemorySpace.SMEM
    full = lambda shape: pl.BlockSpec(shape, lambda s, i: tuple(0 for _ in shape))
    outs = pl.pallas_call(
        _tc_body,
        grid=(2, NB),
        in_specs=[
            full((B, D)),
            full((B, D)),
            full((2, B)),
            full((2, B)),
            full((2, B)),
            pl.BlockSpec(memory_space=smem),
        ],
        out_specs=[pl.BlockSpec((1, 1), lambda s, i: (0, 0), memory_space=smem)] * 7,
        out_shape=[_scal] * 7,
        scratch_shapes=[pltpu.VMEM((D, B), jnp.float32),
                        pltpu.SMEM((8,), jnp.float32)],
    )(image_features, text_features, tau2, s2, b2, e0)
    loss, taui, taut, twim, twtm, twmax, twmin = outs

    return (loss[0, 0], taui[0, 0], taut[0, 0], jnp.float32(ETA_INIT),
            twim[0, 0], twtm[0, 0], twmax[0, 0], twmin[0, 0])


# speedup vs baseline: 1.0991x; 1.0130x over previous
import functools

import jax
import jax.numpy as jnp
from jax import lax
from jax.experimental import pallas as pl
from jax.experimental.pallas import tpu as pltpu
from jax.experimental.pallas import tpu_sc as plsc

B = 2048
D = 256
R = 1024
NB = B // R
GAMMA = 0.8
EPS = 1e-14
RHO = 8.0
GRAD_CLIP = 5.0
ETA_INIT = 1e-05

try:
    _SC_INFO = plsc.get_sparse_core_info()
    _NC, _NS = _SC_INFO.num_cores, _SC_INFO.num_subcores
except ValueError:
    _NC, _NS = 2, 16
_NW = _NC * _NS
_BPW = B // _NW

_vec2 = jax.ShapeDtypeStruct((2, B), jnp.float32)


@functools.partial(
    pl.kernel,
    mesh=plsc.VectorSubcoreMesh(core_axis_name="c", subcore_axis_name="s",
                                num_cores=_NC, num_subcores=_NS),
    out_type=[_vec2] * 3,
    scratch_types=[
        pltpu.VMEM((_BPW,), jnp.int32),
        pltpu.VMEM((_BPW,), jnp.int32),
        pltpu.VMEM((_BPW,), jnp.float32),
        pltpu.VMEM((_BPW,), jnp.float32),
        pltpu.VMEM((_BPW,), jnp.float32),
        pltpu.VMEM((_BPW,), jnp.float32),
        pltpu.VMEM((_BPW,), jnp.float32),
        pltpu.VMEM((_BPW,), jnp.float32),
        pltpu.SemaphoreType.DMA,
    ],
)
def _gather6(img_ids, txt_ids, tau_i_t, s_i_t, b_i_t, tau_t_t, s_t_t, b_t_t,
             o_tau, o_s, o_b, idx_i, idx_t, b0, b1, b2, b3, b4, b5, sem):
    wid = lax.axis_index("s") * _NC + lax.axis_index("c")
    base = wid * _BPW
    pltpu.sync_copy(img_ids.at[pl.ds(base, _BPW)], idx_i)
    pltpu.sync_copy(txt_ids.at[pl.ds(base, _BPW)], idx_t)
    plan = ((0, idx_i, tau_i_t, o_tau, b0), (0, idx_i, s_i_t, o_s, b1),
            (0, idx_i, b_i_t, o_b, b2), (1, idx_t, tau_t_t, o_tau, b3),
            (1, idx_t, s_t_t, o_s, b4), (1, idx_t, b_t_t, o_b, b5))
    copies = [pltpu.async_copy(table.at[idx], buf, sem)
              for row, idx, table, out, buf in plan]
    for c, (row, idx, table, out, buf) in zip(copies, plan):
        c.wait()
        pltpu.sync_copy(buf, out.at[row, pl.ds(base, _BPW)])


def _tc_body(img_ref, txt_ref, tau_ref, s_ref, b_ref, e0_ref,
             loss_ref, taui_ref, taut_ref, twim_ref, twtm_ref,
             twmax_ref, twmin_ref, OT_s, acc):
    s = pl.program_id(0)
    i = pl.program_id(1)
    ii = i * R
    is_img = s == 0

    @pl.when(i == 0)
    def _build_ot():
        def _t_img():
            return txt_ref[...].T

        def _t_txt():
            return img_ref[...].T

        OT_s[...] = lax.cond(is_img, _t_img, _t_txt)

    def _img_side():
        return (img_ref[pl.ds(ii, R), :],
                tau_ref[0, pl.ds(ii, R)], s_ref[0, pl.ds(ii, R)],
                b_ref[0, pl.ds(ii, R)])

    def _txt_side():
        return (txt_ref[pl.ds(ii, R), :],
                tau_ref[1, pl.ds(ii, R)], s_ref[1, pl.ds(ii, R)],
                b_ref[1, pl.ds(ii, R)])

    feat, tau, s_old, b_old = lax.cond(is_img, _img_side, _txt_side)
    S2 = jnp.dot(feat, OT_s[...], preferred_element_type=jnp.float32)
    col = lax.broadcasted_iota(jnp.int32, (R, B), 1)
    row = lax.broadcasted_iota(jnp.int32, (R, B), 0)
    is_diag = col == ii + row
    d = jnp.sum(jnp.where(is_diag, S2, 0.0), axis=1)
    rtau = 1.0 / tau
    m = jnp.max(S2, axis=1)
    b_new = jnp.maximum(b_old, (m - d) * rtau)
    diffs = S2 - d[:, None]
    E = jnp.exp(diffs * rtau[:, None] - b_new[:, None])
    E = jnp.where(is_diag, 0.0, E)
    g = jnp.sum(E, axis=1)
    P1 = jnp.sum(E * diffs, axis=1)
    ema = (1.0 - GAMMA) * s_old * jnp.exp(b_old - b_new) + GAMMA * g
    e0 = e0_ref[0, 0]
    sI = e0 * g + (1.0 - e0) * ema
    sIc = jnp.maximum(sI, EPS)
    rs = 1.0 / sIc
    loss_rows = P1 * rs
    wid_rows = loss_rows * rtau
    tw = jnp.log(sIc / (B - 1)) + b_new + RHO - wid_rows
    tw = jnp.clip(tw, -GRAD_CLIP, GRAD_CLIP)

    blk_loss = jnp.sum(loss_rows)
    blk_twsum = jnp.sum(tw)
    blk_twmax = jnp.max(tw)
    blk_twmin = jnp.min(tw)
    blk_tau = jnp.sum(tau)

    @pl.when((s == 0) & (i == 0))
    def _init():
        acc[0] = blk_loss
        acc[1] = blk_twsum
        acc[2] = 0.0
        acc[3] = blk_twmax
        acc[4] = blk_twmin
        acc[5] = blk_tau
        acc[6] = 0.0

    @pl.when((s != 0) | (i != 0))
    def _accum():
        acc[0] += blk_loss
        acc[1 + s] += blk_twsum
        acc[5 + s] += blk_tau

        @pl.when(s == 0)
        def _mm():
            acc[3] = jnp.maximum(acc[3], blk_twmax)
            acc[4] = jnp.minimum(acc[4], blk_twmin)

    @pl.when((s == 1) & (i == NB - 1))
    def _final():
        invB = jnp.float32(1.0 / B)
        loss_ref[0, 0] = acc[0] * invB
        taui_ref[0, 0] = acc[5] * invB
        taut_ref[0, 0] = acc[6] * invB
        twim_ref[0, 0] = acc[1] * invB
        twtm_ref[0, 0] = acc[2] * invB
        twmax_ref[0, 0] = acc[3]
        twmin_ref[0, 0] = acc[4]


_scal = jax.ShapeDtypeStruct((1, 1), jnp.float32)


def kernel(image_features, text_features, image_ids, text_ids, epoch, max_epoch,
           s_I, s_T, b_I, b_T, u_I, u_T, tau_I, tau_T, mask_neg):
    tau2, s2, b2 = _gather6(image_ids, text_ids, tau_I, s_I, b_I, tau_T, s_T, b_T)

    e0 = (jnp.asarray(epoch) == 0).astype(jnp.float32).reshape(1, 1)

    smem = pltpu.MemorySpace.SMEM
    full = lambda shape: pl.BlockSpec(shape, lambda s, i: tuple(0 for _ in shape))
    outs = pl.pallas_call(
        _tc_body,
        grid=(2, NB),
        in_specs=[
            full((B, D)),
            full((B, D)),
            full((2, B)),
            full((2, B)),
            full((2, B)),
            pl.BlockSpec(memory_space=smem),
        ],
        out_specs=[pl.BlockSpec((1, 1), lambda s, i: (0, 0), memory_space=smem)] * 7,
        out_shape=[_scal] * 7,
        scratch_shapes=[pltpu.VMEM((D, B), jnp.float32),
                        pltpu.SMEM((8,), jnp.float32)],
    )(image_features, text_features, tau2, s2, b2, e0)
    loss, taui, taut, twim, twtm, twmax, twmin = outs

    return (loss[0, 0], taui[0, 0], taut[0, 0], jnp.float32(ETA_INIT),
            twim[0, 0], twtm[0, 0], twmax[0, 0], twmin[0, 0])


# speedup vs baseline: 1.1747x; 1.0688x over previous
import functools

import jax
import jax.numpy as jnp
from jax import lax
from jax.experimental import pallas as pl
from jax.experimental.pallas import tpu as pltpu
from jax.experimental.pallas import tpu_sc as plsc

B = 2048
D = 256
R = 2048
NB = B // R
GAMMA = 0.8
EPS = 1e-14
RHO = 8.0
GRAD_CLIP = 5.0
ETA_INIT = 1e-05

try:
    _SC_INFO = plsc.get_sparse_core_info()
    _NC, _NS = _SC_INFO.num_cores, _SC_INFO.num_subcores
except ValueError:
    _NC, _NS = 2, 16
_NW = _NC * _NS
_BPW = B // _NW

_vec2 = jax.ShapeDtypeStruct((2, B), jnp.float32)


@functools.partial(
    pl.kernel,
    mesh=plsc.VectorSubcoreMesh(core_axis_name="c", subcore_axis_name="s",
                                num_cores=_NC, num_subcores=_NS),
    out_type=[_vec2] * 3,
    scratch_types=[
        pltpu.VMEM((_BPW,), jnp.int32),
        pltpu.VMEM((_BPW,), jnp.int32),
        pltpu.VMEM((_BPW,), jnp.float32),
        pltpu.VMEM((_BPW,), jnp.float32),
        pltpu.VMEM((_BPW,), jnp.float32),
        pltpu.VMEM((_BPW,), jnp.float32),
        pltpu.VMEM((_BPW,), jnp.float32),
        pltpu.VMEM((_BPW,), jnp.float32),
        pltpu.SemaphoreType.DMA,
    ],
)
def _gather6(img_ids, txt_ids, tau_i_t, s_i_t, b_i_t, tau_t_t, s_t_t, b_t_t,
             o_tau, o_s, o_b, idx_i, idx_t, b0, b1, b2, b3, b4, b5, sem):
    wid = lax.axis_index("s") * _NC + lax.axis_index("c")
    base = wid * _BPW
    pltpu.sync_copy(img_ids.at[pl.ds(base, _BPW)], idx_i)
    pltpu.sync_copy(txt_ids.at[pl.ds(base, _BPW)], idx_t)
    plan = ((0, idx_i, tau_i_t, o_tau, b0), (0, idx_i, s_i_t, o_s, b1),
            (0, idx_i, b_i_t, o_b, b2), (1, idx_t, tau_t_t, o_tau, b3),
            (1, idx_t, s_t_t, o_s, b4), (1, idx_t, b_t_t, o_b, b5))
    copies = [pltpu.async_copy(table.at[idx], buf, sem)
              for row, idx, table, out, buf in plan]
    for c, (row, idx, table, out, buf) in zip(copies, plan):
        c.wait()
        pltpu.sync_copy(buf, out.at[row, pl.ds(base, _BPW)])


def _tc_body(img_ref, txt_ref, tau_ref, s_ref, b_ref, e0_ref,
             loss_ref, taui_ref, taut_ref, twim_ref, twtm_ref,
             twmax_ref, twmin_ref, OT_s, acc):
    s = pl.program_id(0)
    i = pl.program_id(1)
    ii = i * R
    is_img = s == 0

    @pl.when(i == 0)
    def _build_ot():
        def _t_img():
            return txt_ref[...].T

        def _t_txt():
            return img_ref[...].T

        OT_s[...] = lax.cond(is_img, _t_img, _t_txt)

    def _img_side():
        return (img_ref[pl.ds(ii, R), :],
                tau_ref[0, pl.ds(ii, R)], s_ref[0, pl.ds(ii, R)],
                b_ref[0, pl.ds(ii, R)])

    def _txt_side():
        return (txt_ref[pl.ds(ii, R), :],
                tau_ref[1, pl.ds(ii, R)], s_ref[1, pl.ds(ii, R)],
                b_ref[1, pl.ds(ii, R)])

    feat, tau, s_old, b_old = lax.cond(is_img, _img_side, _txt_side)
    S2 = jnp.dot(feat, OT_s[...], preferred_element_type=jnp.float32)
    col = lax.broadcasted_iota(jnp.int32, (R, B), 1)
    row = lax.broadcasted_iota(jnp.int32, (R, B), 0)
    is_diag = col == ii + row
    d = jnp.sum(jnp.where(is_diag, S2, 0.0), axis=1)
    rtau = 1.0 / tau
    m = jnp.max(S2, axis=1)
    b_new = jnp.maximum(b_old, (m - d) * rtau)
    diffs = S2 - d[:, None]
    E = jnp.exp(diffs * rtau[:, None] - b_new[:, None])
    E = jnp.where(is_diag, 0.0, E)
    g = jnp.sum(E, axis=1)
    P1 = jnp.sum(E * diffs, axis=1)
    ema = (1.0 - GAMMA) * s_old * jnp.exp(b_old - b_new) + GAMMA * g
    e0 = e0_ref[0, 0]
    sI = e0 * g + (1.0 - e0) * ema
    sIc = jnp.maximum(sI, EPS)
    rs = 1.0 / sIc
    loss_rows = P1 * rs
    wid_rows = loss_rows * rtau
    tw = jnp.log(sIc / (B - 1)) + b_new + RHO - wid_rows
    tw = jnp.clip(tw, -GRAD_CLIP, GRAD_CLIP)

    blk_loss = jnp.sum(loss_rows)
    blk_twsum = jnp.sum(tw)
    blk_twmax = jnp.max(tw)
    blk_twmin = jnp.min(tw)
    blk_tau = jnp.sum(tau)

    @pl.when((s == 0) & (i == 0))
    def _init():
        acc[0] = blk_loss
        acc[1] = blk_twsum
        acc[2] = 0.0
        acc[3] = blk_twmax
        acc[4] = blk_twmin
        acc[5] = blk_tau
        acc[6] = 0.0

    @pl.when((s != 0) | (i != 0))
    def _accum():
        acc[0] += blk_loss
        acc[1 + s] += blk_twsum
        acc[5 + s] += blk_tau

        @pl.when(s == 0)
        def _mm():
            acc[3] = jnp.maximum(acc[3], blk_twmax)
            acc[4] = jnp.minimum(acc[4], blk_twmin)

    @pl.when((s == 1) & (i == NB - 1))
    def _final():
        invB = jnp.float32(1.0 / B)
        loss_ref[0, 0] = acc[0] * invB
        taui_ref[0, 0] = acc[5] * invB
        taut_ref[0, 0] = acc[6] * invB
        twim_ref[0, 0] = acc[1] * invB
        twtm_ref[0, 0] = acc[2] * invB
        twmax_ref[0, 0] = acc[3]
        twmin_ref[0, 0] = acc[4]


_scal = jax.ShapeDtypeStruct((1, 1), jnp.float32)


def kernel(image_features, text_features, image_ids, text_ids, epoch, max_epoch,
           s_I, s_T, b_I, b_T, u_I, u_T, tau_I, tau_T, mask_neg):
    tau2, s2, b2 = _gather6(image_ids, text_ids, tau_I, s_I, b_I, tau_T, s_T, b_T)

    e0 = (jnp.asarray(epoch) == 0).astype(jnp.float32).reshape(1, 1)

    smem = pltpu.MemorySpace.SMEM
    full = lambda shape: pl.BlockSpec(shape, lambda s, i: tuple(0 for _ in shape))
    outs = pl.pallas_call(
        _tc_body,
        grid=(2, NB),
        in_specs=[
            full((B, D)),
            full((B, D)),
            full((2, B)),
            full((2, B)),
            full((2, B)),
            pl.BlockSpec(memory_space=smem),
        ],
        out_specs=[pl.BlockSpec((1, 1), lambda s, i: (0, 0), memory_space=smem)] * 7,
        out_shape=[_scal] * 7,
        scratch_shapes=[pltpu.VMEM((D, B), jnp.float32),
                        pltpu.SMEM((8,), jnp.float32)],
    )(image_features, text_features, tau2, s2, b2, e0)
    loss, taui, taut, twim, twtm, twmax, twmin = outs

    return (loss[0, 0], taui[0, 0], taut[0, 0], jnp.float32(ETA_INIT),
            twim[0, 0], twtm[0, 0], twmax[0, 0], twmin[0, 0])


# speedup vs baseline: 1.1839x; 1.0079x over previous
import functools

import jax
import jax.numpy as jnp
from jax import lax
from jax.experimental import pallas as pl
from jax.experimental.pallas import tpu as pltpu
from jax.experimental.pallas import tpu_sc as plsc

B = 2048
D = 256
R = 2048
NB = B // R
GAMMA = 0.8
EPS = 1e-14
RHO = 8.0
GRAD_CLIP = 5.0
ETA_INIT = 1e-05

try:
    _SC_INFO = plsc.get_sparse_core_info()
    _NC, _NS = _SC_INFO.num_cores, _SC_INFO.num_subcores
except ValueError:
    _NC, _NS = 2, 16
_NW = _NC * _NS
_BPW = B // _NW

_vec2 = jax.ShapeDtypeStruct((2, B), jnp.float32)


@functools.partial(
    pl.kernel,
    mesh=plsc.VectorSubcoreMesh(core_axis_name="c", subcore_axis_name="s",
                                num_cores=_NC, num_subcores=_NS),
    out_type=[_vec2] * 3,
    scratch_types=[
        pltpu.VMEM((_BPW,), jnp.int32),
        pltpu.VMEM((_BPW,), jnp.int32),
        pltpu.VMEM((_BPW,), jnp.float32),
        pltpu.VMEM((_BPW,), jnp.float32),
        pltpu.VMEM((_BPW,), jnp.float32),
        pltpu.VMEM((_BPW,), jnp.float32),
        pltpu.VMEM((_BPW,), jnp.float32),
        pltpu.VMEM((_BPW,), jnp.float32),
        pltpu.SemaphoreType.DMA,
        pltpu.SemaphoreType.DMA,
    ],
)
def _gather6(img_ids, txt_ids, tau_i_t, s_i_t, b_i_t, tau_t_t, s_t_t, b_t_t,
             o_tau, o_s, o_b, idx_i, idx_t, b0, b1, b2, b3, b4, b5, sem, sem2):
    wid = lax.axis_index("s") * _NC + lax.axis_index("c")
    base = wid * _BPW
    ci = pltpu.async_copy(img_ids.at[pl.ds(base, _BPW)], idx_i, sem)
    ct = pltpu.async_copy(txt_ids.at[pl.ds(base, _BPW)], idx_t, sem)
    ci.wait()
    ct.wait()
    plan = ((0, idx_i, tau_i_t, o_tau, b0), (0, idx_i, s_i_t, o_s, b1),
            (0, idx_i, b_i_t, o_b, b2), (1, idx_t, tau_t_t, o_tau, b3),
            (1, idx_t, s_t_t, o_s, b4), (1, idx_t, b_t_t, o_b, b5))
    copies = [pltpu.async_copy(table.at[idx], buf, sem)
              for row, idx, table, out, buf in plan]
    stores = []
    for c, (row, idx, table, out, buf) in zip(copies, plan):
        c.wait()
        stores.append(pltpu.async_copy(buf, out.at[row, pl.ds(base, _BPW)], sem2))
    for st in stores:
        st.wait()


def _tc_body(img_ref, txt_ref, tau_ref, s_ref, b_ref, e0_ref,
             loss_ref, taui_ref, taut_ref, twim_ref, twtm_ref,
             twmax_ref, twmin_ref, OT_s, acc):
    s = pl.program_id(0)
    i = pl.program_id(1)
    ii = i * R
    is_img = s == 0

    @pl.when(i == 0)
    def _build_ot():
        def _t_img():
            return txt_ref[...].T

        def _t_txt():
            return img_ref[...].T

        OT_s[...] = lax.cond(is_img, _t_img, _t_txt)

    def _img_side():
        return (img_ref[pl.ds(ii, R), :],
                tau_ref[0, pl.ds(ii, R)], s_ref[0, pl.ds(ii, R)],
                b_ref[0, pl.ds(ii, R)])

    def _txt_side():
        return (txt_ref[pl.ds(ii, R), :],
                tau_ref[1, pl.ds(ii, R)], s_ref[1, pl.ds(ii, R)],
                b_ref[1, pl.ds(ii, R)])

    feat, tau, s_old, b_old = lax.cond(is_img, _img_side, _txt_side)
    S2 = jnp.dot(feat, OT_s[...], preferred_element_type=jnp.float32)
    col = lax.broadcasted_iota(jnp.int32, (R, B), 1)
    row = lax.broadcasted_iota(jnp.int32, (R, B), 0)
    is_diag = col == ii + row
    d = jnp.sum(jnp.where(is_diag, S2, 0.0), axis=1)
    rtau = 1.0 / tau
    m = jnp.max(S2, axis=1)
    b_new = jnp.maximum(b_old, (m - d) * rtau)
    diffs = S2 - d[:, None]
    E = jnp.exp(diffs * rtau[:, None] - b_new[:, None])
    E = jnp.where(is_diag, 0.0, E)
    g = jnp.sum(E, axis=1)
    P1 = jnp.sum(E * diffs, axis=1)
    ema = (1.0 - GAMMA) * s_old * jnp.exp(b_old - b_new) + GAMMA * g
    e0 = e0_ref[0, 0]
    sI = e0 * g + (1.0 - e0) * ema
    sIc = jnp.maximum(sI, EPS)
    rs = 1.0 / sIc
    loss_rows = P1 * rs
    wid_rows = loss_rows * rtau
    tw = jnp.log(sIc / (B - 1)) + b_new + RHO - wid_rows
    tw = jnp.clip(tw, -GRAD_CLIP, GRAD_CLIP)

    blk_loss = jnp.sum(loss_rows)
    blk_twsum = jnp.sum(tw)
    blk_twmax = jnp.max(tw)
    blk_twmin = jnp.min(tw)
    blk_tau = jnp.sum(tau)

    @pl.when((s == 0) & (i == 0))
    def _init():
        acc[0] = blk_loss
        acc[1] = blk_twsum
        acc[2] = 0.0
        acc[3] = blk_twmax
        acc[4] = blk_twmin
        acc[5] = blk_tau
        acc[6] = 0.0

    @pl.when((s != 0) | (i != 0))
    def _accum():
        acc[0] += blk_loss
        acc[1 + s] += blk_twsum
        acc[5 + s] += blk_tau

        @pl.when(s == 0)
        def _mm():
            acc[3] = jnp.maximum(acc[3], blk_twmax)
            acc[4] = jnp.minimum(acc[4], blk_twmin)

    @pl.when((s == 1) & (i == NB - 1))
    def _final():
        invB = jnp.float32(1.0 / B)
        loss_ref[0, 0] = acc[0] * invB
        taui_ref[0, 0] = acc[5] * invB
        taut_ref[0, 0] = acc[6] * invB
        twim_ref[0, 0] = acc[1] * invB
        twtm_ref[0, 0] = acc[2] * invB
        twmax_ref[0, 0] = acc[3]
        twmin_ref[0, 0] = acc[4]


_scal = jax.ShapeDtypeStruct((1, 1), jnp.float32)


def kernel(image_features, text_features, image_ids, text_ids, epoch, max_epoch,
           s_I, s_T, b_I, b_T, u_I, u_T, tau_I, tau_T, mask_neg):
    tau2, s2, b2 = _gather6(image_ids, text_ids, tau_I, s_I, b_I, tau_T, s_T, b_T)

    e0 = (jnp.asarray(epoch) == 0).astype(jnp.float32).reshape(1, 1)

    smem = pltpu.MemorySpace.SMEM
    full = lambda shape: pl.BlockSpec(shape, lambda s, i: tuple(0 for _ in shape))
    outs = pl.pallas_call(
        _tc_body,
        grid=(2, NB),
        in_specs=[
            full((B, D)),
            full((B, D)),
            full((2, B)),
            full((2, B)),
            full((2, B)),
            pl.BlockSpec(memory_space=smem),
        ],
        out_specs=[pl.BlockSpec((1, 1), lambda s, i: (0, 0), memory_space=smem)] * 7,
        out_shape=[_scal] * 7,
        scratch_shapes=[pltpu.VMEM((D, B), jnp.float32),
                        pltpu.SMEM((8,), jnp.float32)],
    )(image_features, text_features, tau2, s2, b2, e0)
    loss, taui, taut, twim, twtm, twmax, twmin = outs

    return (loss[0, 0], taui[0, 0], taut[0, 0], jnp.float32(ETA_INIT),
            twim[0, 0], twtm[0, 0], twmax[0, 0], twmin[0, 0])


# speedup vs baseline: 1.2004x; 1.0139x over previous
import functools

import jax
import jax.numpy as jnp
from jax import lax
from jax.experimental import pallas as pl
from jax.experimental.pallas import tpu as pltpu
from jax.experimental.pallas import tpu_sc as plsc

B = 2048
D = 256
R = 2048
NB = B // R
GAMMA = 0.8
EPS = 1e-14
RHO = 8.0
GRAD_CLIP = 5.0
ETA_INIT = 1e-05

try:
    _SC_INFO = plsc.get_sparse_core_info()
    _NC, _NS = _SC_INFO.num_cores, _SC_INFO.num_subcores
except ValueError:
    _NC, _NS = 2, 16
_NW = _NC * _NS
_BPW = B // _NW

_vec2 = jax.ShapeDtypeStruct((2, B), jnp.float32)


@functools.partial(
    pl.kernel,
    mesh=plsc.VectorSubcoreMesh(core_axis_name="c", subcore_axis_name="s",
                                num_cores=_NC, num_subcores=_NS),
    out_type=[_vec2] * 3,
    scratch_types=[
        pltpu.VMEM((_BPW,), jnp.int32),
        pltpu.VMEM((_BPW,), jnp.int32),
        pltpu.VMEM((_BPW,), jnp.float32),
        pltpu.VMEM((_BPW,), jnp.float32),
        pltpu.VMEM((_BPW,), jnp.float32),
        pltpu.VMEM((_BPW,), jnp.float32),
        pltpu.VMEM((_BPW,), jnp.float32),
        pltpu.VMEM((_BPW,), jnp.float32),
        pltpu.SemaphoreType.DMA,
        pltpu.SemaphoreType.DMA,
    ],
)
def _gather6(img_ids, txt_ids, tau_i_t, s_i_t, b_i_t, tau_t_t, s_t_t, b_t_t,
             o_tau, o_s, o_b, idx_i, idx_t, b0, b1, b2, b3, b4, b5, sem, sem2):
    wid = lax.axis_index("s") * _NC + lax.axis_index("c")
    base = wid * _BPW
    ci = pltpu.async_copy(img_ids.at[pl.ds(base, _BPW)], idx_i, sem)
    ct = pltpu.async_copy(txt_ids.at[pl.ds(base, _BPW)], idx_t, sem)
    ci.wait()
    ct.wait()
    plan = ((0, idx_i, tau_i_t, o_tau, b0), (0, idx_i, s_i_t, o_s, b1),
            (0, idx_i, b_i_t, o_b, b2), (1, idx_t, tau_t_t, o_tau, b3),
            (1, idx_t, s_t_t, o_s, b4), (1, idx_t, b_t_t, o_b, b5))
    copies = [pltpu.async_copy(table.at[idx], buf, sem)
              for row, idx, table, out, buf in plan]
    stores = []
    for c, (row, idx, table, out, buf) in zip(copies, plan):
        c.wait()
        stores.append(pltpu.async_copy(buf, out.at[row, pl.ds(base, _BPW)], sem2))
    for st in stores:
        st.wait()


def _transpose2_body(img_ref, txt_ref, ot_ref):
    s = pl.program_id(0)

    def _t_img():
        return txt_ref[...].T

    def _t_txt():
        return img_ref[...].T

    ot_ref[0] = lax.cond(s == 0, _t_img, _t_txt)


def _tc_body(img_ref, txt_ref, ot_ref, tau_ref, s_ref, b_ref, e0_ref,
             loss_ref, taui_ref, taut_ref, twim_ref, twtm_ref,
             twmax_ref, twmin_ref, acc):
    s = pl.program_id(0)
    i = pl.program_id(1)
    ii = i * R
    is_img = s == 0

    def _img_side():
        return (img_ref[pl.ds(ii, R), :],
                tau_ref[0, pl.ds(ii, R)], s_ref[0, pl.ds(ii, R)],
                b_ref[0, pl.ds(ii, R)])

    def _txt_side():
        return (txt_ref[pl.ds(ii, R), :],
                tau_ref[1, pl.ds(ii, R)], s_ref[1, pl.ds(ii, R)],
                b_ref[1, pl.ds(ii, R)])

    feat, tau, s_old, b_old = lax.cond(is_img, _img_side, _txt_side)
    S2 = jnp.dot(feat, ot_ref[0], preferred_element_type=jnp.float32)
    col = lax.broadcasted_iota(jnp.int32, (R, B), 1)
    row = lax.broadcasted_iota(jnp.int32, (R, B), 0)
    is_diag = col == ii + row
    d = jnp.sum(jnp.where(is_diag, S2, 0.0), axis=1)
    rtau = 1.0 / tau
    m = jnp.max(S2, axis=1)
    b_new = jnp.maximum(b_old, (m - d) * rtau)
    diffs = S2 - d[:, None]
    E = jnp.exp(diffs * rtau[:, None] - b_new[:, None])
    E = jnp.where(is_diag, 0.0, E)
    g = jnp.sum(E, axis=1)
    P1 = jnp.sum(E * diffs, axis=1)
    ema = (1.0 - GAMMA) * s_old * jnp.exp(b_old - b_new) + GAMMA * g
    e0 = e0_ref[0, 0]
    sI = e0 * g + (1.0 - e0) * ema
    sIc = jnp.maximum(sI, EPS)
    rs = 1.0 / sIc
    loss_rows = P1 * rs
    wid_rows = loss_rows * rtau
    tw = jnp.log(sIc / (B - 1)) + b_new + RHO - wid_rows
    tw = jnp.clip(tw, -GRAD_CLIP, GRAD_CLIP)

    blk_loss = jnp.sum(loss_rows)
    blk_twsum = jnp.sum(tw)
    blk_twmax = jnp.max(tw)
    blk_twmin = jnp.min(tw)
    blk_tau = jnp.sum(tau)

    @pl.when((s == 0) & (i == 0))
    def _init():
        acc[0] = blk_loss
        acc[1] = blk_twsum
        acc[2] = 0.0
        acc[3] = blk_twmax
        acc[4] = blk_twmin
        acc[5] = blk_tau
        acc[6] = 0.0

    @pl.when((s != 0) | (i != 0))
    def _accum():
        acc[0] += blk_loss
        acc[1 + s] += blk_twsum
        acc[5 + s] += blk_tau

        @pl.when(s == 0)
        def _mm():
            acc[3] = jnp.maximum(acc[3], blk_twmax)
            acc[4] = jnp.minimum(acc[4], blk_twmin)

    @pl.when((s == 1) & (i == NB - 1))
    def _final():
        invB = jnp.float32(1.0 / B)
        loss_ref[0, 0] = acc[0] * invB
        taui_ref[0, 0] = acc[5] * invB
        taut_ref[0, 0] = acc[6] * invB
        twim_ref[0, 0] = acc[1] * invB
        twtm_ref[0, 0] = acc[2] * invB
        twmax_ref[0, 0] = acc[3]
        twmin_ref[0, 0] = acc[4]


_scal = jax.ShapeDtypeStruct((1, 1), jnp.float32)


def kernel(image_features, text_features, image_ids, text_ids, epoch, max_epoch,
           s_I, s_T, b_I, b_T, u_I, u_T, tau_I, tau_T, mask_neg):
    tau2, s2, b2 = _gather6(image_ids, text_ids, tau_I, s_I, b_I, tau_T, s_T, b_T)

    e0 = (jnp.asarray(epoch) == 0).astype(jnp.float32).reshape(1, 1)

    ot = pl.pallas_call(
        _transpose2_body,
        grid=(2,),
        in_specs=[
            pl.BlockSpec((B, D), lambda s: (0, 0)),
            pl.BlockSpec((B, D), lambda s: (0, 0)),
        ],
        out_specs=pl.BlockSpec((1, D, B), lambda s: (s, 0, 0)),
        out_shape=jax.ShapeDtypeStruct((2, D, B), jnp.float32),
    )(image_features, text_features)

    smem = pltpu.MemorySpace.SMEM
    full = lambda shape: pl.BlockSpec(shape, lambda s, i: tuple(0 for _ in shape))
    outs = pl.pallas_call(
        _tc_body,
        grid=(2, NB),
        in_specs=[
            full((B, D)),
            full((B, D)),
            pl.BlockSpec((1, D, B), lambda s, i: (s, 0, 0)),
            full((2, B)),
            full((2, B)),
            full((2, B)),
            pl.BlockSpec(memory_space=smem),
        ],
        out_specs=[pl.BlockSpec((1, 1), lambda s, i: (0, 0), memory_space=smem)] * 7,
        out_shape=[_scal] * 7,
        scratch_shapes=[pltpu.SMEM((8,), jnp.float32)],
    )(image_features, text_features, ot, tau2, s2, b2, e0)
    loss, taui, taut, twim, twtm, twmax, twmin = outs

    return (loss[0, 0], taui[0, 0], taut[0, 0], jnp.float32(ETA_INIT),
            twim[0, 0], twtm[0, 0], twmax[0, 0], twmin[0, 0])
